# baseline (speedup 1.0000x reference)
import jax
import jax.numpy as jnp
from jax.experimental import pallas as pl
from jax.experimental.pallas import tpu as pltpu

_BM = 1024


def _matmul_body(x_ref, w_ref, b_ref, o_ref):
    x = x_ref[...].astype(jnp.bfloat16)
    o_ref[...] = (
        jnp.dot(x, w_ref[...], preferred_element_type=jnp.float32) + b_ref[...]
    )


def kernel(inputs, W, b):
    m, k = inputs.shape
    n = W.shape[1]
    w_bf16 = W.astype(jnp.bfloat16)
    b2 = b.reshape(1, n)
    grid = (m // _BM,)
    return pl.pallas_call(
        _matmul_body,
        grid=grid,
        in_specs=[
            pl.BlockSpec((_BM, k), lambda i: (i, 0)),
            pl.BlockSpec((k, n), lambda i: (0, 0)),
            pl.BlockSpec((1, n), lambda i: (0, 0)),
        ],
        out_specs=pl.BlockSpec((_BM, n), lambda i: (i, 0)),
        out_shape=jax.ShapeDtypeStruct((m, n), jnp.float32),
        compiler_params=pltpu.CompilerParams(
            dimension_semantics=("arbitrary",),
        ),
    )(inputs, w_bf16, b2)


# speedup vs baseline: 1.0262x; 1.0262x over previous
import jax
import jax.numpy as jnp
from jax.experimental import pallas as pl
from jax.experimental.pallas import tpu as pltpu

_BM = 1024
_WCHUNK = 512
_NCHUNK = 4


def _matmul_body(x_ref, w_hbm, b_ref, o_ref, w_bf, stage, sem):
    @pl.when(pl.program_id(0) == 0)
    def _load_w():
        pltpu.make_async_copy(
            w_hbm.at[:, pl.ds(0, _WCHUNK)], stage.at[0], sem.at[0]
        ).start()
        for c in range(_NCHUNK):
            if c + 1 < _NCHUNK:
                pltpu.make_async_copy(
                    w_hbm.at[:, pl.ds((c + 1) * _WCHUNK, _WCHUNK)],
                    stage.at[(c + 1) % 2],
                    sem.at[(c + 1) % 2],
                ).start()
            pltpu.make_async_copy(
                w_hbm.at[:, pl.ds(c * _WCHUNK, _WCHUNK)],
                stage.at[c % 2],
                sem.at[c % 2],
            ).wait()
            w_bf[:, c * _WCHUNK : (c + 1) * _WCHUNK] = stage[c % 2].astype(
                jnp.bfloat16
            )

    x = x_ref[...].astype(jnp.bfloat16)
    o_ref[...] = (
        jnp.dot(x, w_bf[...], preferred_element_type=jnp.float32) + b_ref[...]
    )


def kernel(inputs, W, b):
    m, k = inputs.shape
    n = W.shape[1]
    b2 = b.reshape(1, n)
    grid = (m // _BM,)
    return pl.pallas_call(
        _matmul_body,
        grid=grid,
        in_specs=[
            pl.BlockSpec((_BM, k), lambda i: (i, 0)),
            pl.BlockSpec(memory_space=pltpu.MemorySpace.HBM),
            pl.BlockSpec((1, n), lambda i: (0, 0)),
        ],
        out_specs=pl.BlockSpec((_BM, n), lambda i: (i, 0)),
        out_shape=jax.ShapeDtypeStruct((m, n), jnp.float32),
        scratch_shapes=[
            pltpu.VMEM((k, n), jnp.bfloat16),
            pltpu.VMEM((2, k, _WCHUNK), jnp.float32),
            pltpu.SemaphoreType.DMA((2,)),
        ],
        compiler_params=pltpu.CompilerParams(
            dimension_semantics=("arbitrary",),
        ),
    )(inputs, W, b2)
